# baseline (speedup 1.0000x reference)
import jax
import jax.numpy as jnp
from jax import lax
from jax.experimental import pallas as pl
from jax.experimental.pallas import tpu as pltpu
from jax.experimental.pallas import tpu_sc as plsc

BATCH = 4096
N_FIELDS = 26
D = 32
_PREP_W = 4096
_PREP_GRID = -(-1000000 // _PREP_W)
VROWS = _PREP_GRID * 1024
NC = 2
NS = 16
NW = NC * NS
BPW = BATCH // NW
CB = 16
NCHUNK = BPW // CB
GROUPS = BPW // 16


def _sc_body(feat_hbm, feate_hbm, emb_hbm, lin_hbm, bi_hbm, linsum_hbm,
             idx_v, idx4_v, feat2_v, rows_a, rows_b, linflat_v,
             bi_v, linsum_v, sem_e, sem_l):
    wid = lax.axis_index("s") * NC + lax.axis_index("c")

    pltpu.sync_copy(feat_hbm.at[wid], idx_v)
    pltpu.sync_copy(feate_hbm.at[wid], feat2_v)

    lin_copies = [
        pltpu.async_copy(lin_hbm.at[idx_v.at[f]],
                         linflat_v.at[pl.ds(f * 128, 128)], sem_l)
        for f in range(N_FIELDS)
    ]

    for f in range(N_FIELDS):
        for g in range(GROUPS):
            v = idx_v[f, pl.ds(g * 16, 16)]
            vrow = (lax.shift_right_logical(v, 12) * 1024
                    + (lax.shift_right_logical(v, 9) & 7) * 128
                    + (v & 127))
            idx4_v[f, pl.ds(g * 16, 16)] = vrow

    def fire_chunk(c, buf):
        return [
            pltpu.async_copy(emb_hbm.at[idx4_v.at[f, pl.ds(c * CB, CB)]],
                             buf.at[pl.ds(f * CB, CB)], sem_e)
            for f in range(N_FIELDS)
        ]

    bufs = (rows_a, rows_b)
    pending = fire_chunk(0, bufs[0])
    for c in range(NCHUNK):
        nxt = fire_chunk(c + 1, bufs[(c + 1) % 2]) if c + 1 < NCHUNK else []
        for cp in pending:
            cp.wait()
        pending = nxt
        buf = bufs[c % 2]

        def row_body(i, _):
            z = jnp.zeros((16,), jnp.float32)
            s_lo, s_hi, q_lo, q_hi = z, z, z, z
            r1 = feat2_v[c * CB + i, pl.ds(0, 16)]
            r2 = feat2_v[c * CB + i, pl.ds(N_FIELDS - 16, 16)]
            o1 = (lax.shift_right_logical(r1, 7) & 3) * D
            o2 = (lax.shift_right_logical(r2, 7) & 3) * D
            for f in range(N_FIELDS):
                o = o1[f] if f < 16 else o2[f - (N_FIELDS - 16)]
                v_lo = buf[f * CB + i, pl.ds(o, 16)]
                v_hi = buf[f * CB + i, pl.ds(o + 16, 16)]
                s_lo = s_lo + v_lo
                s_hi = s_hi + v_hi
                q_lo = q_lo + v_lo * v_lo
                q_hi = q_hi + v_hi * v_hi
            bi_v[c * CB + i, pl.ds(0, 16)] = 0.5 * (s_lo * s_lo - q_lo)
            bi_v[c * CB + i, pl.ds(16, 16)] = 0.5 * (s_hi * s_hi - q_hi)
            return 0
        lax.fori_loop(0, CB, row_body, 0)

    for cp in lin_copies:
        cp.wait()

    for g in range(GROUPS):
        acc = jnp.zeros((16,), jnp.float32)
        for f in range(N_FIELDS):
            acc = acc + linflat_v[pl.ds(f * 128 + g * 16, 16)]
        linsum_v[pl.ds(g * 16, 16)] = acc

    pltpu.sync_copy(bi_v, bi_hbm.at[pl.ds(wid * BPW, BPW)])
    pltpu.sync_copy(linsum_v, linsum_hbm.at[pl.ds(wid * BPW, BPW)])


@jax.jit
def _sc_gather_reduce(features_t, features_e, emb4, lin_table):
    mesh = plsc.VectorSubcoreMesh(core_axis_name="c", subcore_axis_name="s")
    return pl.kernel(
        _sc_body,
        out_type=[
            jax.ShapeDtypeStruct((BATCH, D), jnp.float32),
            jax.ShapeDtypeStruct((BATCH,), jnp.float32),
        ],
        mesh=mesh,
        compiler_params=pltpu.CompilerParams(use_tc_tiling_on_sc=False),
        scratch_types=[
            pltpu.VMEM((N_FIELDS, 128), jnp.int32),
            pltpu.VMEM((N_FIELDS, 128), jnp.int32),
            pltpu.VMEM((BPW, N_FIELDS), jnp.int32),
            pltpu.VMEM((N_FIELDS * CB, 128), jnp.float32),
            pltpu.VMEM((N_FIELDS * CB, 128), jnp.float32),
            pltpu.VMEM((N_FIELDS * 128,), jnp.float32),
            pltpu.VMEM((BPW, D), jnp.float32),
            pltpu.VMEM((BPW,), jnp.float32),
            pltpu.SemaphoreType.DMA,
            pltpu.SemaphoreType.DMA,
        ],
    )(features_t, features_e, emb4, lin_table)


def _prep_body(embt_ref, lint_ref, emb4_ref, linflat_ref):
    for m in range(_PREP_W // 512):
        s = jnp.concatenate(
            [embt_ref[:, pl.ds((4 * m + a) * 128, 128)] for a in range(4)],
            axis=0)
        emb4_ref[pl.ds(m * 128, 128), :] = jnp.transpose(s, (1, 0))
    linflat_ref[...] = lint_ref[0, :]


@jax.jit
def _tc_prep(emb_t, lin_t):
    return pl.pallas_call(
        _prep_body,
        grid=(_PREP_GRID,),
        in_specs=[
            pl.BlockSpec((D, _PREP_W), lambda i: (0, i)),
            pl.BlockSpec((1, _PREP_W), lambda i: (0, i)),
        ],
        out_specs=[
            pl.BlockSpec((_PREP_W // 4, 128), lambda i: (i, 0)),
            pl.BlockSpec((_PREP_W,), lambda i: (i,)),
        ],
        out_shape=[
            jax.ShapeDtypeStruct((VROWS, 128), jnp.float32),
            jax.ShapeDtypeStruct((1000000,), jnp.float32),
        ],
    )(emb_t, lin_t)


def _mlp_body(bi_ref, lin_ref, w1_ref, b1_ref, w2_ref, b2_ref, w3_ref, b3_ref,
              out_ref):
    h = jnp.maximum(
        jnp.dot(bi_ref[...], w1_ref[...],
                preferred_element_type=jnp.float32) + b1_ref[...], 0.0)
    h = jnp.maximum(
        jnp.dot(h, w2_ref[...], preferred_element_type=jnp.float32)
        + b2_ref[...], 0.0)
    deep = jnp.sum(h * w3_ref[...], axis=1)
    out_ref[...] = deep + b3_ref[0, 0] + lin_ref[...]


@jax.jit
def _tc_mlp(bi, linsum, W1, b1, W2, b2, W3, b3):
    return pl.pallas_call(
        _mlp_body,
        out_shape=jax.ShapeDtypeStruct((BATCH,), jnp.float32),
    )(bi, linsum, W1, b1.reshape(1, -1),
      W2, b2.reshape(1, -1), W3.reshape(1, -1), b3.reshape(1, 1))


def kernel(features, emb_table, lin_table, W1, b1, W2, b2, W3, b3):
    features_e = features.astype(jnp.int32).reshape(NW, BPW, N_FIELDS)
    features_t = features_e.transpose(0, 2, 1)
    emb4, lin_flat = _tc_prep(emb_table.T, lin_table.T)
    bi, linsum = _sc_gather_reduce(features_t, features_e, emb4, lin_flat)
    return _tc_mlp(bi, linsum, W1, b1, W2, b2, W3, b3)


# speedup vs baseline: 1.0131x; 1.0131x over previous
import jax
import jax.numpy as jnp
from jax import lax
from jax.experimental import pallas as pl
from jax.experimental.pallas import tpu as pltpu
from jax.experimental.pallas import tpu_sc as plsc

BATCH = 4096
N_FIELDS = 26
D = 32
_PREP_W = 4096
_PREP_GRID = -(-1000000 // _PREP_W)
VROWS = _PREP_GRID * 1024
NC = 2
NS = 16
NW = NC * NS
BPW = BATCH // NW
CB = 16
NCHUNK = BPW // CB
GROUPS = BPW // 16


def _sc_body(feat_hbm, emb_hbm, lin_hbm, bi_hbm, linsum_hbm,
             idx_v, idx4_v, off_v, rows_a, rows_b, linflat_v,
             bi_v, linsum_v, sem_e, sem_l):
    wid = lax.axis_index("s") * NC + lax.axis_index("c")

    pltpu.sync_copy(feat_hbm.at[wid], idx_v)

    lin_copies = [
        pltpu.async_copy(lin_hbm.at[idx_v.at[f]],
                         linflat_v.at[pl.ds(f * 128, 128)], sem_l)
        for f in range(N_FIELDS)
    ]

    for f in range(N_FIELDS):
        for g in range(GROUPS):
            v = idx_v[f, pl.ds(g * 16, 16)]
            vrow = (lax.shift_right_logical(v, 12) * 1024
                    + (lax.shift_right_logical(v, 9) & 7) * 128
                    + (v & 127))
            idx4_v[f, pl.ds(g * 16, 16)] = vrow
            off_v[f, pl.ds(g * 16, 16)] = (lax.shift_right_logical(v, 7) & 3) * D

    def fire_chunk(c, buf):
        return [
            pltpu.async_copy(emb_hbm.at[idx4_v.at[f, pl.ds(c * CB, CB)]],
                             buf.at[pl.ds(f * CB, CB)], sem_e)
            for f in range(N_FIELDS)
        ]

    bufs = (rows_a, rows_b)
    pending = fire_chunk(0, bufs[0])
    for c in range(NCHUNK):
        nxt = fire_chunk(c + 1, bufs[(c + 1) % 2]) if c + 1 < NCHUNK else []
        for cp in pending:
            cp.wait()
        pending = nxt
        buf = bufs[c % 2]

        def row_body(i, _):
            z = jnp.zeros((16,), jnp.float32)
            s_lo, s_hi, q_lo, q_hi = z, z, z, z
            for f in range(N_FIELDS):
                o = off_v[f, pl.ds(c * CB + i, 16)][0]
                v_lo = buf[f * CB + i, pl.ds(o, 16)]
                v_hi = buf[f * CB + i, pl.ds(o + 16, 16)]
                s_lo = s_lo + v_lo
                s_hi = s_hi + v_hi
                q_lo = q_lo + v_lo * v_lo
                q_hi = q_hi + v_hi * v_hi
            bi_v[c * CB + i, pl.ds(0, 16)] = 0.5 * (s_lo * s_lo - q_lo)
            bi_v[c * CB + i, pl.ds(16, 16)] = 0.5 * (s_hi * s_hi - q_hi)
            return 0
        lax.fori_loop(0, CB, row_body, 0)

    for cp in lin_copies:
        cp.wait()

    for g in range(GROUPS):
        acc = jnp.zeros((16,), jnp.float32)
        for f in range(N_FIELDS):
            acc = acc + linflat_v[pl.ds(f * 128 + g * 16, 16)]
        linsum_v[pl.ds(g * 16, 16)] = acc

    pltpu.sync_copy(bi_v, bi_hbm.at[pl.ds(wid * BPW, BPW)])
    pltpu.sync_copy(linsum_v, linsum_hbm.at[pl.ds(wid * BPW, BPW)])


@jax.jit
def _sc_gather_reduce(features_t, emb4, lin_table):
    mesh = plsc.VectorSubcoreMesh(core_axis_name="c", subcore_axis_name="s")
    return pl.kernel(
        _sc_body,
        out_type=[
            jax.ShapeDtypeStruct((BATCH, D), jnp.float32),
            jax.ShapeDtypeStruct((BATCH,), jnp.float32),
        ],
        mesh=mesh,
        compiler_params=pltpu.CompilerParams(use_tc_tiling_on_sc=False),
        scratch_types=[
            pltpu.VMEM((N_FIELDS, 128), jnp.int32),
            pltpu.VMEM((N_FIELDS, 128), jnp.int32),
            pltpu.VMEM((N_FIELDS, 144), jnp.int32),
            pltpu.VMEM((N_FIELDS * CB, 128), jnp.float32),
            pltpu.VMEM((N_FIELDS * CB, 128), jnp.float32),
            pltpu.VMEM((N_FIELDS * 128,), jnp.float32),
            pltpu.VMEM((BPW, D), jnp.float32),
            pltpu.VMEM((BPW,), jnp.float32),
            pltpu.SemaphoreType.DMA,
            pltpu.SemaphoreType.DMA,
        ],
    )(features_t, emb4, lin_table)


def _prep_body(embt_ref, lint_ref, emb4_ref, linflat_ref):
    for m in range(_PREP_W // 512):
        s = jnp.concatenate(
            [embt_ref[:, pl.ds((4 * m + a) * 128, 128)] for a in range(4)],
            axis=0)
        emb4_ref[pl.ds(m * 128, 128), :] = jnp.transpose(s, (1, 0))
    linflat_ref[...] = lint_ref[0, :]


@jax.jit
def _tc_prep(emb_t, lin_t):
    return pl.pallas_call(
        _prep_body,
        grid=(_PREP_GRID,),
        in_specs=[
            pl.BlockSpec((D, _PREP_W), lambda i: (0, i)),
            pl.BlockSpec((1, _PREP_W), lambda i: (0, i)),
        ],
        out_specs=[
            pl.BlockSpec((_PREP_W // 4, 128), lambda i: (i, 0)),
            pl.BlockSpec((_PREP_W,), lambda i: (i,)),
        ],
        out_shape=[
            jax.ShapeDtypeStruct((VROWS, 128), jnp.float32),
            jax.ShapeDtypeStruct((1000000,), jnp.float32),
        ],
    )(emb_t, lin_t)


def _mlp_body(bi_ref, lin_ref, w1_ref, b1_ref, w2_ref, b2_ref, w3_ref, b3_ref,
              out_ref):
    h = jnp.maximum(
        jnp.dot(bi_ref[...], w1_ref[...],
                preferred_element_type=jnp.float32) + b1_ref[...], 0.0)
    h = jnp.maximum(
        jnp.dot(h, w2_ref[...], preferred_element_type=jnp.float32)
        + b2_ref[...], 0.0)
    deep = jnp.sum(h * w3_ref[...], axis=1)
    out_ref[...] = deep + b3_ref[0, 0] + lin_ref[...]


@jax.jit
def _tc_mlp(bi, linsum, W1, b1, W2, b2, W3, b3):
    return pl.pallas_call(
        _mlp_body,
        out_shape=jax.ShapeDtypeStruct((BATCH,), jnp.float32),
    )(bi, linsum, W1, b1.reshape(1, -1),
      W2, b2.reshape(1, -1), W3.reshape(1, -1), b3.reshape(1, 1))


def kernel(features, emb_table, lin_table, W1, b1, W2, b2, W3, b3):
    features_t = features.astype(jnp.int32).reshape(
        NW, BPW, N_FIELDS).transpose(0, 2, 1)
    emb4, lin_flat = _tc_prep(emb_table.T, lin_table.T)
    bi, linsum = _sc_gather_reduce(features_t, emb4, lin_flat)
    return _tc_mlp(bi, linsum, W1, b1, W2, b2, W3, b3)
